# baseline (speedup 1.0000x reference)
import functools

import jax
import jax.numpy as jnp
from jax import lax
from jax.experimental import pallas as pl
from jax.experimental.pallas import tpu as pltpu
from jax.experimental.pallas import tpu_sc as plsc

N = 10000
E = 320000
D = 128

NC = 2
NS = 16
NW = NC * NS
HE = E // 2
WP = HE // NW
K = 80
CHF = WP // K
TK = WP - CHF * K
NB = 4
NP = 10240
RPT = NP // NS

_HI = jax.lax.Precision.HIGHEST


def _make_gather_body(piece):
    e0 = piece * HE

    def body_fn(p_hbm, q_hbm, src_hbm, dst_hbm, s_hbm,
                idxs, idxd, idxts, idxtd, prow, qrow,
                si0, si1, si2, si3, sg0, sg1, sg2, sg3,
                st0, st1, st2, st3):
        sem_i = (si0, si1, si2, si3)
        sem_g = (sg0, sg1, sg2, sg3)
        sem_st = (st0, st1, st2, st3)
        cid = lax.axis_index("c")
        sid = lax.axis_index("s")
        wid = cid * NS + sid
        w0 = e0 + wid * WP
        o0 = wid * WP

        def fire_idx(j, b):
            pltpu.async_copy(src_hbm.at[pl.ds(w0 + j * K, K)], idxs.at[b],
                             sem_i[b])
            pltpu.async_copy(dst_hbm.at[pl.ds(w0 + j * K, K)], idxd.at[b],
                             sem_i[b])

        def fire_gather(b):
            pltpu.async_copy(p_hbm.at[idxs.at[b]], prow.at[b], sem_g[b])
            pltpu.async_copy(q_hbm.at[idxd.at[b]], qrow.at[b], sem_g[b])

        def wait_idx(b):
            pltpu.make_async_copy(src_hbm.at[pl.ds(w0, K)], idxs.at[b],
                                  sem_i[b]).wait()
            pltpu.make_async_copy(dst_hbm.at[pl.ds(w0, K)], idxd.at[b],
                                  sem_i[b]).wait()

        def wait_gather(b):
            pltpu.make_async_copy(p_hbm.at[pl.ds(0, K)], prow.at[b],
                                  sem_g[b]).wait()
            pltpu.make_async_copy(q_hbm.at[pl.ds(0, K)], qrow.at[b],
                                  sem_g[b]).wait()

        def wait_store(b):
            pltpu.make_async_copy(prow.at[b], s_hbm.at[pl.ds(o0, K)],
                                  sem_st[b]).wait()

        fire_idx(0, 0)
        fire_idx(1, 1)
        fire_idx(2, 2)
        wait_idx(0)
        fire_gather(0)
        wait_idx(1)
        fire_gather(1)

        def iter_j(j, b):
            bn = (b + 2) % NB
            bi = (b + 3) % NB

            @pl.when(j >= 2)
            def _():
                wait_store(bn)

            @pl.when(j + 3 < CHF)
            def _():
                fire_idx(j + 3, bi)

            @pl.when(j + 2 < CHF)
            def _():
                wait_idx(bn)
                fire_gather(bn)

            wait_gather(b)

            def add_row(r, cc):
                for u in range(D // 16):
                    sl = pl.ds(u * 16, 16)
                    prow[b, r, sl] = prow[b, r, sl] + qrow[b, r, sl]
                return cc

            lax.fori_loop(0, K, add_row, 0)
            pltpu.async_copy(prow.at[b], s_hbm.at[pl.ds(o0 + j * K, K)],
                             sem_st[b])

        def outer(t, carry):
            for u in range(NB):
                j = t * NB + u

                @pl.when(j < CHF)
                def _():
                    iter_j(j, u)

            return carry

        lax.fori_loop(0, -(-CHF // NB), outer, 0)
        wait_store((CHF - 2) % NB)
        wait_store((CHF - 1) % NB)

        tb = 0
        pltpu.sync_copy(src_hbm.at[pl.ds(w0 + CHF * K, TK)], idxts)
        pltpu.sync_copy(dst_hbm.at[pl.ds(w0 + CHF * K, TK)], idxtd)
        cp1 = pltpu.async_copy(p_hbm.at[idxts], prow.at[tb, pl.ds(0, TK)],
                               sem_g[tb])
        cp2 = pltpu.async_copy(q_hbm.at[idxtd], qrow.at[tb, pl.ds(0, TK)],
                               sem_g[tb])
        cp1.wait()
        cp2.wait()

        def add_row_t(r, cc):
            for u in range(D // 16):
                sl = pl.ds(u * 16, 16)
                prow[tb, r, sl] = prow[tb, r, sl] + qrow[tb, r, sl]
            return cc

        lax.fori_loop(0, TK, add_row_t, 0)
        pltpu.sync_copy(prow.at[tb, pl.ds(0, TK)],
                        s_hbm.at[pl.ds(o0 + CHF * K, TK)])

    return body_fn


def _make_scatter_body(piece):
    e0 = piece * HE

    def body_fn(eupd_hbm, dst_hbm, zeros_hbm, out_hbm,
                idxd, idxt, rows, agg_sp,
                sr0, sr1, sr2, sr3, ss0, ss1, ss2, ss3):
        sem_r = (sr0, sr1, sr2, sr3)
        sem_s = (ss0, ss1, ss2, ss3)
        cid = lax.axis_index("c")
        sid = lax.axis_index("s")
        wid = cid * NS + sid
        w0 = e0 + wid * WP
        o0 = wid * WP

        pltpu.sync_copy(zeros_hbm.at[pl.ds(sid * RPT, RPT)],
                        agg_sp.at[pl.ds(sid * RPT, RPT)])
        plsc.subcore_barrier()

        def fire(j, b):
            pltpu.async_copy(dst_hbm.at[pl.ds(w0 + j * K, K)], idxd.at[b],
                             sem_r[b])
            pltpu.async_copy(eupd_hbm.at[pl.ds(o0 + j * K, K)], rows.at[b],
                             sem_r[b])

        def wait_rows(b):
            pltpu.make_async_copy(dst_hbm.at[pl.ds(w0, K)], idxd.at[b],
                                  sem_r[b]).wait()
            pltpu.make_async_copy(eupd_hbm.at[pl.ds(o0, K)], rows.at[b],
                                  sem_r[b]).wait()

        def wait_scatter(b):
            pltpu.make_async_copy(rows.at[b], agg_sp.at[pl.ds(0, K)],
                                  sem_s[b]).wait()

        fire(0, 0)
        fire(1, 1)

        def iter_j(j, b):
            bn = (b + 2) % NB

            @pl.when(j >= 2)
            def _():
                wait_scatter(bn)

            @pl.when(j + 2 < CHF)
            def _():
                fire(j + 2, bn)

            wait_rows(b)
            pltpu.async_copy(rows.at[b], agg_sp.at[idxd.at[b]], sem_s[b],
                             add=True)

        def outer(t, carry):
            for u in range(NB):
                j = t * NB + u

                @pl.when(j < CHF)
                def _():
                    iter_j(j, u)

            return carry

        lax.fori_loop(0, -(-CHF // NB), outer, 0)
        wait_scatter((CHF - 2) % NB)
        wait_scatter((CHF - 1) % NB)

        tb = 0
        pltpu.sync_copy(dst_hbm.at[pl.ds(w0 + CHF * K, TK)], idxt)
        pltpu.sync_copy(eupd_hbm.at[pl.ds(o0 + CHF * K, TK)],
                        rows.at[tb, pl.ds(0, TK)])
        pltpu.sync_copy(rows.at[tb, pl.ds(0, TK)], agg_sp.at[idxt], add=True)

        plsc.subcore_barrier()
        pltpu.sync_copy(agg_sp.at[pl.ds(sid * RPT, RPT)],
                        out_hbm.at[cid, pl.ds(sid * RPT, RPT)])

    return body_fn


@functools.lru_cache(maxsize=None)
def _sc_kernels():
    mesh = plsc.VectorSubcoreMesh(
        core_axis_name="c", subcore_axis_name="s",
        num_cores=NC, num_subcores=NS,
    )
    gathers = []
    scatters = []
    for piece in (0, 1):
        gathers.append(pl.kernel(
            _make_gather_body(piece),
            out_type=jax.ShapeDtypeStruct((HE, D), jnp.float32),
            mesh=mesh,
            scratch_types=(
                [
                    pltpu.VMEM((NB, K), jnp.int32),
                    pltpu.VMEM((NB, K), jnp.int32),
                    pltpu.VMEM((TK,), jnp.int32),
                    pltpu.VMEM((TK,), jnp.int32),
                    pltpu.VMEM((NB, K, D), jnp.float32),
                    pltpu.VMEM((NB, K, D), jnp.float32),
                ]
                + [pltpu.SemaphoreType.DMA] * 12
            ),
        ))
        scatters.append(pl.kernel(
            _make_scatter_body(piece),
            out_type=jax.ShapeDtypeStruct((NC, NP, D), jnp.float32),
            mesh=mesh,
            scratch_types=(
                [
                    pltpu.VMEM((NB, K), jnp.int32),
                    pltpu.VMEM((TK,), jnp.int32),
                    pltpu.VMEM((NB, K, D), jnp.float32),
                    pltpu.VMEM_SHARED((NP, D), jnp.float32),
                ]
                + [pltpu.SemaphoreType.DMA] * 8
            ),
        ))
    return gathers, scatters


def _proj_body(v_ref, wa_ref, wb_ref, p_ref, q_ref):
    v = v_ref[...]
    p_ref[...] = lax.dot_general(v, wa_ref[...], (((1,), (0,)), ((), ())),
                                 precision=_HI)
    q_ref[...] = lax.dot_general(v, wb_ref[...], (((1,), (0,)), ((), ())),
                                 precision=_HI)


def _edge_body0(s_ref, e_ref, wc_ref, be_ref, enew_ref, eupd_ref):
    e = e_ref[...]
    acc = (s_ref[...]
           + lax.dot_general(e, wc_ref[...], (((1,), (0,)), ((), ())))
           + be_ref[...])
    upd = jnp.maximum(acc, 0.0)
    eupd_ref[...] = upd
    enew_ref[...] = e + upd


def _edge_body1(s_ref, e_ref, wc_ref, be_ref, alias_ref, enew_ref, eupd_ref):
    _edge_body0(s_ref, e_ref, wc_ref, be_ref, enew_ref, eupd_ref)


def _vtx_body(v_ref, a0_ref, a1_ref, a2_ref, a3_ref,
              wva_ref, wvb_ref, bv_ref, out_ref):
    v = v_ref[...]
    agg = (a0_ref[...] + a1_ref[...]) + (a2_ref[...] + a3_ref[...])
    h = (lax.dot_general(v, wva_ref[...], (((1,), (0,)), ((), ())),
                         precision=_HI)
         + lax.dot_general(agg, wvb_ref[...], (((1,), (0,)), ((), ())),
                           precision=_HI)
         + bv_ref[...])
    out_ref[...] = v + jnp.maximum(h, 0.0)


_VB = 2000
_EB = 8000


def _proj(v, wa, wb):
    return pl.pallas_call(
        _proj_body,
        grid=(N // _VB,),
        in_specs=[
            pl.BlockSpec((_VB, D), lambda i: (i, 0)),
            pl.BlockSpec((D, D), lambda i: (0, 0)),
            pl.BlockSpec((D, D), lambda i: (0, 0)),
        ],
        out_specs=[
            pl.BlockSpec((_VB, D), lambda i: (i, 0)),
            pl.BlockSpec((_VB, D), lambda i: (i, 0)),
        ],
        out_shape=[
            jax.ShapeDtypeStruct((N, D), jnp.float32),
            jax.ShapeDtypeStruct((N, D), jnp.float32),
        ],
    )(v, wa, wb)


def _edge(piece, s, e, wc, be2d, alias=None):
    nblk = HE // _EB
    off = piece * nblk
    in_specs = [
        pl.BlockSpec((_EB, D), lambda i: (i, 0)),
        pl.BlockSpec((_EB, D), lambda i, o=off: (i + o, 0)),
        pl.BlockSpec((D, D), lambda i: (0, 0)),
        pl.BlockSpec((1, D), lambda i: (0, 0)),
    ]
    args = [s, e, wc, be2d]
    kwargs = {}
    body = _edge_body0
    if alias is not None:
        in_specs.append(pl.BlockSpec(memory_space=pl.ANY))
        args.append(alias)
        kwargs["input_output_aliases"] = {4: 0}
        body = _edge_body1
    return pl.pallas_call(
        body,
        grid=(nblk,),
        in_specs=in_specs,
        out_specs=[
            pl.BlockSpec((_EB, D), lambda i, o=off: (i + o, 0)),
            pl.BlockSpec((_EB, D), lambda i: (i, 0)),
        ],
        out_shape=[
            jax.ShapeDtypeStruct((E, D), jnp.float32),
            jax.ShapeDtypeStruct((HE, D), jnp.float32),
        ],
        **kwargs,
    )(*args)


def _vtx(v, a0, a1, a2, a3, wva, wvb, bv2d):
    return pl.pallas_call(
        _vtx_body,
        grid=(N // _VB,),
        in_specs=[
            pl.BlockSpec((_VB, D), lambda i: (i, 0)),
            pl.BlockSpec((_VB, D), lambda i: (i, 0)),
            pl.BlockSpec((_VB, D), lambda i: (i, 0)),
            pl.BlockSpec((_VB, D), lambda i: (i, 0)),
            pl.BlockSpec((_VB, D), lambda i: (i, 0)),
            pl.BlockSpec((D, D), lambda i: (0, 0)),
            pl.BlockSpec((D, D), lambda i: (0, 0)),
            pl.BlockSpec((1, D), lambda i: (0, 0)),
        ],
        out_specs=pl.BlockSpec((_VB, D), lambda i: (i, 0)),
        out_shape=jax.ShapeDtypeStruct((N, D), jnp.float32),
    )(v, a0, a1, a2, a3, wva, wvb, bv2d)


def kernel(vertex_features, edge_features, edge_index,
           We0, be0, Wv0, bv0, We1, be1, Wv1, bv1):
    src = edge_index[0]
    dst = edge_index[1]
    zeros = jnp.zeros((NP, D), jnp.float32)

    gathers, scatters = _sc_kernels()
    v, e = vertex_features, edge_features
    for (We, be, Wv, bv) in ((We0, be0, Wv0, bv0), (We1, be1, Wv1, bv1)):
        p, q = _proj(v, We[:D], We[D:2 * D])
        wc = We[2 * D:]
        be2d = be.reshape(1, D)
        s0 = gathers[0](p, q, src, dst)
        s1 = gathers[1](p, q, src, dst)
        enew0, eupd0 = _edge(0, s0, e, wc, be2d)
        aggp0 = scatters[0](eupd0, dst, zeros)
        enew, eupd1 = _edge(1, s1, e, wc, be2d, alias=enew0)
        aggp1 = scatters[1](eupd1, dst, zeros)
        v = _vtx(v, aggp0[0], aggp0[1], aggp1[0], aggp1[1],
                 Wv[:D], Wv[D:], bv.reshape(1, D))
        e = enew
    return v, e


# speedup vs baseline: 1.0321x; 1.0321x over previous
import functools

import jax
import jax.numpy as jnp
from jax import lax
from jax.experimental import pallas as pl
from jax.experimental.pallas import tpu as pltpu
from jax.experimental.pallas import tpu_sc as plsc

N = 10000
E = 320000
D = 128

NC = 2
NS = 16
NW = NC * NS
HE = E // 2
WP = HE // NW
K = 80
CHF = WP // K
TK = WP - CHF * K
NB = 4
NP = 10112
RPT = NP // NS
DW = D // 2
KS = 64
CHS = WP // KS
TKS = WP - CHS * KS

_HI = jax.lax.Precision.HIGHEST


def _make_gather_body():
    def body_fn(p_hbm, q_hbm, src_hbm, dst_hbm, s_hbm,
                idxs, idxd, idxts, idxtd, prow, qrow,
                si0, si1, si2, si3, sg0, sg1, sg2, sg3,
                st0, st1, st2, st3):
        sem_i = (si0, si1, si2, si3)
        sem_g = (sg0, sg1, sg2, sg3)
        sem_st = (st0, st1, st2, st3)
        cid = lax.axis_index("c")
        sid = lax.axis_index("s")
        wid = cid * NS + sid
        w0 = wid * WP
        o0 = wid * WP

        def fire_idx(j, b):
            pltpu.async_copy(src_hbm.at[pl.ds(w0 + j * K, K)], idxs.at[b],
                             sem_i[b])
            pltpu.async_copy(dst_hbm.at[pl.ds(w0 + j * K, K)], idxd.at[b],
                             sem_i[b])

        def fire_gather(b):
            pltpu.async_copy(p_hbm.at[idxs.at[b]], prow.at[b], sem_g[b])
            pltpu.async_copy(q_hbm.at[idxd.at[b]], qrow.at[b], sem_g[b])

        def wait_idx(b):
            pltpu.make_async_copy(src_hbm.at[pl.ds(w0, K)], idxs.at[b],
                                  sem_i[b]).wait()
            pltpu.make_async_copy(dst_hbm.at[pl.ds(w0, K)], idxd.at[b],
                                  sem_i[b]).wait()

        def wait_gather(b):
            pltpu.make_async_copy(p_hbm.at[pl.ds(0, K)], prow.at[b],
                                  sem_g[b]).wait()
            pltpu.make_async_copy(q_hbm.at[pl.ds(0, K)], qrow.at[b],
                                  sem_g[b]).wait()

        def wait_store(b):
            pltpu.make_async_copy(prow.at[b], s_hbm.at[pl.ds(o0, K)],
                                  sem_st[b]).wait()

        fire_idx(0, 0)
        fire_idx(1, 1)
        fire_idx(2, 2)
        wait_idx(0)
        fire_gather(0)
        wait_idx(1)
        fire_gather(1)

        def iter_j(j, b):
            bn = (b + 2) % NB
            bi = (b + 3) % NB

            @pl.when(j >= 2)
            def _():
                wait_store(bn)

            @pl.when(j + 3 < CHF)
            def _():
                fire_idx(j + 3, bi)

            @pl.when(j + 2 < CHF)
            def _():
                wait_idx(bn)
                fire_gather(bn)

            wait_gather(b)

            def add_row(r, cc):
                for u in range(D // 16):
                    sl = pl.ds(u * 16, 16)
                    prow[b, r, sl] = prow[b, r, sl] + qrow[b, r, sl]
                return cc

            lax.fori_loop(0, K, add_row, 0)
            pltpu.async_copy(prow.at[b], s_hbm.at[pl.ds(o0 + j * K, K)],
                             sem_st[b])

        def outer(t, carry):
            for u in range(NB):
                j = t * NB + u

                @pl.when(j < CHF)
                def _():
                    iter_j(j, u)

            return carry

        lax.fori_loop(0, -(-CHF // NB), outer, 0)
        wait_store((CHF - 2) % NB)
        wait_store((CHF - 1) % NB)

        tb = 0
        pltpu.sync_copy(src_hbm.at[pl.ds(w0 + CHF * K, TK)], idxts)
        pltpu.sync_copy(dst_hbm.at[pl.ds(w0 + CHF * K, TK)], idxtd)
        cp1 = pltpu.async_copy(p_hbm.at[idxts], prow.at[tb, pl.ds(0, TK)],
                               sem_g[tb])
        cp2 = pltpu.async_copy(q_hbm.at[idxtd], qrow.at[tb, pl.ds(0, TK)],
                               sem_g[tb])
        cp1.wait()
        cp2.wait()

        def add_row_t(r, cc):
            for u in range(D // 16):
                sl = pl.ds(u * 16, 16)
                prow[tb, r, sl] = prow[tb, r, sl] + qrow[tb, r, sl]
            return cc

        lax.fori_loop(0, TK, add_row_t, 0)
        pltpu.sync_copy(prow.at[tb, pl.ds(0, TK)],
                        s_hbm.at[pl.ds(o0 + CHF * K, TK)])

    return body_fn


def _make_scatter_body():
    def body_fn(eupd_hbm, dst_hbm, init_hbm, out_hbm,
                idxd, idxt, rows, frows, agg_sp,
                sr0, sr1, sr2, sr3, ss0, ss1, ss2, ss3):
        sem_r = (sr0, sr1, sr2, sr3)
        sem_s = (ss0, ss1, ss2, ss3)
        cid = lax.axis_index("c")
        sid = lax.axis_index("s")
        wid = cid * NS + sid
        w0 = wid * WP
        o0 = wid * WP

        pltpu.sync_copy(init_hbm.at[cid, pl.ds(sid * RPT, RPT)],
                        agg_sp.at[pl.ds(sid * RPT, RPT)])
        plsc.subcore_barrier()

        mh = jnp.int32(-65536)
        sh = jnp.int32(16)

        def fire(j, b):
            pltpu.async_copy(dst_hbm.at[pl.ds(w0 + j * KS, KS)], idxd.at[b],
                             sem_r[b])
            pltpu.async_copy(eupd_hbm.at[pl.ds(o0 + j * KS, KS)], rows.at[b],
                             sem_r[b])

        def wait_rows(b):
            pltpu.make_async_copy(dst_hbm.at[pl.ds(w0, KS)], idxd.at[b],
                                  sem_r[b]).wait()
            pltpu.make_async_copy(eupd_hbm.at[pl.ds(o0, KS)], rows.at[b],
                                  sem_r[b]).wait()

        def wait_scatter(b):
            pltpu.make_async_copy(frows.at[b % 2], agg_sp.at[pl.ds(0, KS)],
                                  sem_s[b]).wait()

        fire(0, 0)
        fire(1, 1)

        def iter_j(j, b):
            bn = (b + 2) % NB

            @pl.when(j >= 2)
            def _():
                wait_scatter(bn)

            @pl.when(j + 2 < CHS)
            def _():
                fire(j + 2, bn)

            wait_rows(b)

            fb = b % 2

            def unpack_row(r, cc):
                for u in range(DW // 16):
                    sl = pl.ds(u * 16, 16)
                    x = rows[b, r, sl]
                    frows[fb, r, sl] = lax.bitcast_convert_type(
                        lax.shift_left(x, sh), jnp.float32)
                    frows[fb, r, pl.ds(DW + u * 16, 16)] = (
                        lax.bitcast_convert_type(x & mh, jnp.float32))
                return cc

            lax.fori_loop(0, KS, unpack_row, 0)
            pltpu.async_copy(frows.at[fb], agg_sp.at[idxd.at[b]], sem_s[b],
                             add=True)

        def outer(t, carry):
            for u in range(NB):
                j = t * NB + u

                @pl.when(j < CHS)
                def _():
                    iter_j(j, u)

            return carry

        lax.fori_loop(0, -(-CHS // NB), outer, 0)
        wait_scatter((CHS - 2) % NB)
        wait_scatter((CHS - 1) % NB)

        tb = 0
        pltpu.sync_copy(dst_hbm.at[pl.ds(w0 + CHS * KS, TKS)], idxt)
        pltpu.sync_copy(eupd_hbm.at[pl.ds(o0 + CHS * KS, TKS)],
                        rows.at[tb, pl.ds(0, TKS)])

        def unpack_row_t(r, cc):
            for u in range(DW // 16):
                sl = pl.ds(u * 16, 16)
                x = rows[tb, r, sl]
                frows[tb, r, sl] = lax.bitcast_convert_type(
                    lax.shift_left(x, sh), jnp.float32)
                frows[tb, r, pl.ds(DW + u * 16, 16)] = (
                    lax.bitcast_convert_type(x & mh, jnp.float32))
            return cc

        lax.fori_loop(0, TKS, unpack_row_t, 0)
        pltpu.sync_copy(frows.at[tb, pl.ds(0, TKS)], agg_sp.at[idxt], add=True)

        plsc.subcore_barrier()
        pltpu.sync_copy(agg_sp.at[pl.ds(sid * RPT, RPT)],
                        out_hbm.at[cid, pl.ds(sid * RPT, RPT)])

    return body_fn


@functools.lru_cache(maxsize=None)
def _sc_kernels():
    mesh = plsc.VectorSubcoreMesh(
        core_axis_name="c", subcore_axis_name="s",
        num_cores=NC, num_subcores=NS,
    )
    gather = pl.kernel(
            _make_gather_body(),
            out_type=jax.ShapeDtypeStruct((HE, D), jnp.float32),
            mesh=mesh,
            scratch_types=(
                [
                    pltpu.VMEM((NB, K), jnp.int32),
                    pltpu.VMEM((NB, K), jnp.int32),
                    pltpu.VMEM((TK,), jnp.int32),
                    pltpu.VMEM((TK,), jnp.int32),
                    pltpu.VMEM((NB, K, D), jnp.float32),
                    pltpu.VMEM((NB, K, D), jnp.float32),
                ]
                + [pltpu.SemaphoreType.DMA] * 12
            ),
        )
    scatter = pl.kernel(
            _make_scatter_body(),
            out_type=jax.ShapeDtypeStruct((NC, NP, D), jnp.float32),
            mesh=mesh,
            scratch_types=(
                [
                    pltpu.VMEM((NB, KS), jnp.int32),
                    pltpu.VMEM((TKS,), jnp.int32),
                    pltpu.VMEM((NB, KS, DW), jnp.int32),
                    pltpu.VMEM((2, KS, D), jnp.float32),
                    pltpu.VMEM_SHARED((NP, D), jnp.float32),
                ]
                + [pltpu.SemaphoreType.DMA] * 8
            ),
        )
    return gather, scatter


def _proj_body(v_ref, wa_ref, wb_ref, p_ref, q_ref):
    v = v_ref[...]
    p_ref[...] = lax.dot_general(v, wa_ref[...], (((1,), (0,)), ((), ())),
                                 precision=_HI)
    q_ref[...] = lax.dot_general(v, wb_ref[...], (((1,), (0,)), ((), ())),
                                 precision=_HI)


def _bf16_pack(x):
    lo = x[:, :DW].astype(jnp.bfloat16).astype(jnp.float32)
    hi = x[:, DW:].astype(jnp.bfloat16).astype(jnp.float32)
    lob = lax.shift_right_logical(lax.bitcast_convert_type(lo, jnp.uint32),
                                  jnp.uint32(16))
    hib = lax.bitcast_convert_type(hi, jnp.uint32) & jnp.uint32(0xFFFF0000)
    return lax.bitcast_convert_type(lob | hib, jnp.int32)


def _edge_body0(s_ref, e_ref, wc_ref, be_ref, enew_ref, eupd_ref):
    e = e_ref[...]
    acc = (s_ref[...]
           + lax.dot_general(e, wc_ref[...], (((1,), (0,)), ((), ())))
           + be_ref[...])
    upd = jnp.maximum(acc, 0.0)
    eupd_ref[...] = _bf16_pack(upd)
    enew_ref[...] = e + upd


def _edge_body1(s_ref, e_ref, wc_ref, be_ref, alias_ref, enew_ref, eupd_ref):
    _edge_body0(s_ref, e_ref, wc_ref, be_ref, enew_ref, eupd_ref)


def _vtx_body(v_ref, a0_ref, a1_ref,
              wva_ref, wvb_ref, bv_ref, out_ref):
    v = v_ref[...]
    agg = a0_ref[...] + a1_ref[...]
    h = (lax.dot_general(v, wva_ref[...], (((1,), (0,)), ((), ())),
                         precision=_HI)
         + lax.dot_general(agg, wvb_ref[...], (((1,), (0,)), ((), ())),
                           precision=_HI)
         + bv_ref[...])
    out_ref[...] = v + jnp.maximum(h, 0.0)


_VB = 2000
_EB = 8000


def _proj(v, wa, wb):
    return pl.pallas_call(
        _proj_body,
        grid=(N // _VB,),
        in_specs=[
            pl.BlockSpec((_VB, D), lambda i: (i, 0)),
            pl.BlockSpec((D, D), lambda i: (0, 0)),
            pl.BlockSpec((D, D), lambda i: (0, 0)),
        ],
        out_specs=[
            pl.BlockSpec((_VB, D), lambda i: (i, 0)),
            pl.BlockSpec((_VB, D), lambda i: (i, 0)),
        ],
        out_shape=[
            jax.ShapeDtypeStruct((N, D), jnp.float32),
            jax.ShapeDtypeStruct((N, D), jnp.float32),
        ],
    )(v, wa, wb)


def _edge(piece, s, e, wc, be2d, alias=None):
    nblk = HE // _EB
    off = piece * nblk
    in_specs = [
        pl.BlockSpec((_EB, D), lambda i: (i, 0)),
        pl.BlockSpec((_EB, D), lambda i, o=off: (i + o, 0)),
        pl.BlockSpec((D, D), lambda i: (0, 0)),
        pl.BlockSpec((1, D), lambda i: (0, 0)),
    ]
    args = [s, e, wc, be2d]
    kwargs = {}
    body = _edge_body0
    if alias is not None:
        in_specs.append(pl.BlockSpec(memory_space=pl.ANY))
        args.append(alias)
        kwargs["input_output_aliases"] = {4: 0}
        body = _edge_body1
    return pl.pallas_call(
        body,
        grid=(nblk,),
        in_specs=in_specs,
        out_specs=[
            pl.BlockSpec((_EB, D), lambda i, o=off: (i + o, 0)),
            pl.BlockSpec((_EB, DW), lambda i: (i, 0)),
        ],
        out_shape=[
            jax.ShapeDtypeStruct((E, D), jnp.float32),
            jax.ShapeDtypeStruct((HE, DW), jnp.int32),
        ],
        **kwargs,
    )(*args)


def _vtx(v, a0, a1, wva, wvb, bv2d):
    return pl.pallas_call(
        _vtx_body,
        grid=(N // _VB,),
        in_specs=[
            pl.BlockSpec((_VB, D), lambda i: (i, 0)),
            pl.BlockSpec((_VB, D), lambda i: (i, 0)),
            pl.BlockSpec((_VB, D), lambda i: (i, 0)),
            pl.BlockSpec((D, D), lambda i: (0, 0)),
            pl.BlockSpec((D, D), lambda i: (0, 0)),
            pl.BlockSpec((1, D), lambda i: (0, 0)),
        ],
        out_specs=pl.BlockSpec((_VB, D), lambda i: (i, 0)),
        out_shape=jax.ShapeDtypeStruct((N, D), jnp.float32),
    )(v, a0, a1, wva, wvb, bv2d)


def kernel(vertex_features, edge_features, edge_index,
           We0, be0, Wv0, bv0, We1, be1, Wv1, bv1):
    src = edge_index[0]
    dst = edge_index[1]
    zeros = jnp.zeros((NC, NP, D), jnp.float32)

    src_p = (src[:HE], src[HE:])
    dst_p = (dst[:HE], dst[HE:])
    sc_gather, sc_scatter = _sc_kernels()
    v, e = vertex_features, edge_features
    for (We, be, Wv, bv) in ((We0, be0, Wv0, bv0), (We1, be1, Wv1, bv1)):
        p, q = _proj(v, We[:D], We[D:2 * D])
        wc = We[2 * D:]
        be2d = be.reshape(1, D)
        s0 = sc_gather(p, q, src_p[0], dst_p[0])
        s1 = sc_gather(p, q, src_p[1], dst_p[1])
        enew0, eupd0 = _edge(0, s0, e, wc, be2d)
        aggp0 = sc_scatter(eupd0, dst_p[0], zeros)
        enew, eupd1 = _edge(1, s1, e, wc, be2d, alias=enew0)
        aggp = sc_scatter(eupd1, dst_p[1], aggp0)
        v = _vtx(v, aggp[0], aggp[1], Wv[:D], Wv[D:], bv.reshape(1, D))
        e = enew
    return v, e
